# baseline (speedup 1.0000x reference)
import jax
import jax.numpy as jnp
from jax import lax
from jax.experimental import pallas as pl
from jax.experimental.pallas import tpu as pltpu
from jax.experimental.pallas import tpu_sc as plsc

_NW = 32
_R = 16
_B = 4
_S = 4096
_D = 2048
_CHUNKS = _S // _NW // _R
_T = _CHUNKS * _B
_VECS = _R * (_D // 16)


def _sc_body(x_hbm, pos_hbm, out_hbm,
             p0_v, x0_v, x1_v,
             spos0, sin0, sin1, sout0, sout1):
    wid = lax.axis_index("s") * 2 + lax.axis_index("c")
    s0 = wid * (_S // _NW)
    pbufs = (p0_v,)
    xbufs = (x0_v, x1_v)
    spos = (spos0,)
    sin = (sin0, sin1)
    sout = (sout0, sout1)

    def in_copy(t):
        chunk, b = divmod(t, _B)
        row = b * _S + s0 + chunk * _R
        return pltpu.make_async_copy(
            x_hbm.at[pl.ds(row, _R)], xbufs[t % 2], sin[t % 2])

    def out_copy(t):
        chunk, b = divmod(t, _B)
        row = b * _S + s0 + chunk * _R
        return pltpu.make_async_copy(
            xbufs[t % 2], out_hbm.at[pl.ds(row, _R)], sout[t % 2])

    def pos_copy(chunk):
        return pltpu.make_async_copy(
            pos_hbm.at[pl.ds(s0 + chunk * _R, _R)], pbufs[0], spos[0])

    def compute(t):
        xb = xbufs[t % 2]
        pb = pbufs[0]

        @plsc.parallel_loop(0, _VECS, unroll=8)
        def body(i):
            r = i // (_D // 16)
            c = (i - r * (_D // 16)) * 16
            plsc.addupdate(xb.at[r, pl.ds(c, 16)], pb[r, pl.ds(c, 16)])

    pos_copy(0).start()
    in_copy(0).start()
    for t in range(_T):
        chunk, b = divmod(t, _B)
        in_copy(t).wait()
        if b == 0:
            pos_copy(chunk).wait()
        compute(t)
        out_copy(t).start()
        if b == _B - 1 and chunk + 1 < _CHUNKS:
            pos_copy(chunk + 1).start()
        if t >= 1:
            out_copy(t - 1).wait()
        if t + 1 < _T:
            in_copy(t + 1).start()
    out_copy(_T - 1).wait()


def kernel(x, pos_table):
    B, S, D = x.shape
    x2 = x.reshape(B * S, D)
    mesh = plsc.VectorSubcoreMesh(core_axis_name="c", subcore_axis_name="s")
    out = pl.kernel(
        _sc_body,
        mesh=mesh,
        out_type=jax.ShapeDtypeStruct((B * S, D), x.dtype),
        scratch_types=[
            pltpu.VMEM((_R, _D), jnp.float32),
            pltpu.VMEM((_R, _D), jnp.float32),
            pltpu.VMEM((_R, _D), jnp.float32),
            pltpu.SemaphoreType.DMA,
            pltpu.SemaphoreType.DMA,
            pltpu.SemaphoreType.DMA,
            pltpu.SemaphoreType.DMA,
            pltpu.SemaphoreType.DMA,
        ],
    )(x2, pos_table)
    return out.reshape(B, S, D)


# speedup vs baseline: 1.4557x; 1.4557x over previous
import jax
import jax.numpy as jnp
from jax import lax
from jax.experimental import pallas as pl
from jax.experimental.pallas import tpu as pltpu
from jax.experimental.pallas import tpu_sc as plsc

_NW = 32
_R = 8
_B = 4
_S = 4096
_D = 2048
_CHUNKS = _S // _NW // _R
_T = _CHUNKS * _B
_VECS = _R * (_D // 16)


def _sc_body(x_hbm, pos_hbm, out_hbm,
             p0_v, p1_v, x0_v, x1_v, x2_v, x3_v,
             spos0, spos1, sin0, sin1, sin2, sin3, sout0, sout1, sout2, sout3):
    wid = lax.axis_index("s") * 2 + lax.axis_index("c")
    s0 = wid * (_S // _NW)
    pbufs = (p0_v, p1_v)
    xbufs = (x0_v, x1_v, x2_v, x3_v)
    spos = (spos0, spos1)
    sin = (sin0, sin1, sin2, sin3)
    sout = (sout0, sout1, sout2, sout3)

    def in_copy(t):
        chunk, b = divmod(t, _B)
        row = b * _S + s0 + chunk * _R
        return pltpu.make_async_copy(
            x_hbm.at[pl.ds(row, _R)], xbufs[t % 4], sin[t % 4])

    def out_copy(t):
        chunk, b = divmod(t, _B)
        row = b * _S + s0 + chunk * _R
        return pltpu.make_async_copy(
            xbufs[t % 4], out_hbm.at[pl.ds(row, _R)], sout[t % 4])

    def pos_copy(chunk):
        return pltpu.make_async_copy(
            pos_hbm.at[pl.ds(s0 + chunk * _R, _R)], pbufs[chunk % 2],
            spos[chunk % 2])

    def compute(t):
        xb = xbufs[t % 4]
        pb = pbufs[(t // _B) % 2]

        @plsc.parallel_loop(0, _VECS, unroll=8)
        def body(i):
            r = i // (_D // 16)
            c = (i - r * (_D // 16)) * 16
            plsc.addupdate(xb.at[r, pl.ds(c, 16)], pb[r, pl.ds(c, 16)])

    pos_copy(0).start()
    pos_copy(1).start()
    in_copy(0).start()
    in_copy(1).start()
    in_copy(2).start()
    for t in range(_T):
        chunk, b = divmod(t, _B)
        in_copy(t).wait()
        if b == 0:
            pos_copy(chunk).wait()
        compute(t)
        out_copy(t).start()
        if b == _B - 1 and chunk + 2 < _CHUNKS:
            pos_copy(chunk + 2).start()
        if t >= 1:
            out_copy(t - 1).wait()
        if t + 3 < _T:
            in_copy(t + 3).start()
    out_copy(_T - 1).wait()


def kernel(x, pos_table):
    B, S, D = x.shape
    x2 = x.reshape(B * S, D)
    mesh = plsc.VectorSubcoreMesh(core_axis_name="c", subcore_axis_name="s")
    out = pl.kernel(
        _sc_body,
        mesh=mesh,
        out_type=jax.ShapeDtypeStruct((B * S, D), x.dtype),
        scratch_types=[
            pltpu.VMEM((_R, _D), jnp.float32),
            pltpu.VMEM((_R, _D), jnp.float32),
            pltpu.VMEM((_R, _D), jnp.float32),
            pltpu.VMEM((_R, _D), jnp.float32),
            pltpu.VMEM((_R, _D), jnp.float32),
            pltpu.VMEM((_R, _D), jnp.float32),
            pltpu.SemaphoreType.DMA,
            pltpu.SemaphoreType.DMA,
            pltpu.SemaphoreType.DMA,
            pltpu.SemaphoreType.DMA,
            pltpu.SemaphoreType.DMA,
            pltpu.SemaphoreType.DMA,
            pltpu.SemaphoreType.DMA,
            pltpu.SemaphoreType.DMA,
            pltpu.SemaphoreType.DMA,
            pltpu.SemaphoreType.DMA,
        ],
    )(x2, pos_table)
    return out.reshape(B, S, D)


# speedup vs baseline: 1.4640x; 1.0057x over previous
import jax
import jax.numpy as jnp
from jax import lax
from jax.experimental import pallas as pl
from jax.experimental.pallas import tpu as pltpu
from jax.experimental.pallas import tpu_sc as plsc

_NW = 32
_R = 8
_B = 4
_S = 4096
_D = 2048
_CHUNKS = _S // _NW // _R
_T = _CHUNKS * _B
_VECS = _R * (_D // 16)


def _sc_body(x_hbm, pos_hbm, out_hbm,
             p0_v, p1_v, x0_v, x1_v, x2_v, x3_v, x4_v,
             spos0, spos1, sin0, sin1, sin2, sin3, sin4, sout0, sout1, sout2, sout3, sout4):
    wid = lax.axis_index("s") * 2 + lax.axis_index("c")
    s0 = wid * (_S // _NW)
    pbufs = (p0_v, p1_v)
    xbufs = (x0_v, x1_v, x2_v, x3_v, x4_v)
    spos = (spos0, spos1)
    sin = (sin0, sin1, sin2, sin3, sin4)
    sout = (sout0, sout1, sout2, sout3, sout4)

    def in_copy(t):
        chunk, b = divmod(t, _B)
        row = b * _S + s0 + chunk * _R
        return pltpu.make_async_copy(
            x_hbm.at[pl.ds(row, _R)], xbufs[t % 5], sin[t % 5])

    def out_copy(t):
        chunk, b = divmod(t, _B)
        row = b * _S + s0 + chunk * _R
        return pltpu.make_async_copy(
            xbufs[t % 5], out_hbm.at[pl.ds(row, _R)], sout[t % 5])

    def pos_copy(chunk):
        return pltpu.make_async_copy(
            pos_hbm.at[pl.ds(s0 + chunk * _R, _R)], pbufs[chunk % 2],
            spos[chunk % 2])

    def compute(t):
        xb = xbufs[t % 5]
        pb = pbufs[(t // _B) % 2]

        @plsc.parallel_loop(0, _VECS, unroll=8)
        def body(i):
            r = i // (_D // 16)
            c = (i - r * (_D // 16)) * 16
            plsc.addupdate(xb.at[r, pl.ds(c, 16)], pb[r, pl.ds(c, 16)])

    pos_copy(0).start()
    pos_copy(1).start()
    in_copy(0).start()
    in_copy(1).start()
    in_copy(2).start()
    in_copy(3).start()
    for t in range(_T):
        chunk, b = divmod(t, _B)
        in_copy(t).wait()
        if b == 0:
            pos_copy(chunk).wait()
        compute(t)
        out_copy(t).start()
        if b == _B - 1 and chunk + 2 < _CHUNKS:
            pos_copy(chunk + 2).start()
        if t >= 1:
            out_copy(t - 1).wait()
        if t + 4 < _T:
            in_copy(t + 4).start()
    out_copy(_T - 1).wait()


def kernel(x, pos_table):
    B, S, D = x.shape
    x2 = x.reshape(B * S, D)
    mesh = plsc.VectorSubcoreMesh(core_axis_name="c", subcore_axis_name="s")
    out = pl.kernel(
        _sc_body,
        mesh=mesh,
        out_type=jax.ShapeDtypeStruct((B * S, D), x.dtype),
        scratch_types=[
            pltpu.VMEM((_R, _D), jnp.float32),
            pltpu.VMEM((_R, _D), jnp.float32),
            pltpu.VMEM((_R, _D), jnp.float32),
            pltpu.VMEM((_R, _D), jnp.float32),
            pltpu.VMEM((_R, _D), jnp.float32),
            pltpu.VMEM((_R, _D), jnp.float32),
            pltpu.VMEM((_R, _D), jnp.float32),
            pltpu.SemaphoreType.DMA,
            pltpu.SemaphoreType.DMA,
            pltpu.SemaphoreType.DMA,
            pltpu.SemaphoreType.DMA,
            pltpu.SemaphoreType.DMA,
            pltpu.SemaphoreType.DMA,
            pltpu.SemaphoreType.DMA,
            pltpu.SemaphoreType.DMA,
            pltpu.SemaphoreType.DMA,
            pltpu.SemaphoreType.DMA,
            pltpu.SemaphoreType.DMA,
            pltpu.SemaphoreType.DMA,
        ],
    )(x2, pos_table)
    return out.reshape(B, S, D)
